# baseline (speedup 1.0000x reference)
import jax
import jax.numpy as jnp
from jax.experimental import pallas as pl
from jax.experimental.pallas import tpu as pltpu

_ROWS_PER_TILE = 2048


def _mod_kernel(idx_ref, emb_hbm, W_ref, b_ref, out_ref, row_vmem, dma_sem):
    i = pl.program_id(0)
    idx = idx_ref[0]

    @pl.when(i == 0)
    def _fetch():
        start = (idx // 2) * 128
        cp = pltpu.make_async_copy(emb_hbm.at[pl.ds(start, 128)], row_vmem, dma_sem)
        cp.start()
        cp.wait()

    row = row_vmem[...]
    code = jnp.where(idx % 2 == 0, row[:64], row[64:])
    mods = jnp.sum(W_ref[...] * code[None, :], axis=1) + b_ref[0, :]
    out_ref[...] = jnp.broadcast_to(mods[None, :], out_ref.shape)


def kernel(coords, patient_idx, emb_table, W, b):
    n = coords.shape[0]
    num_out, code_dim = W.shape
    idx = jnp.asarray(patient_idx, jnp.int32).reshape((1,))
    flat = emb_table.reshape(-1)
    grid = (n // _ROWS_PER_TILE,)
    out = pl.pallas_call(
        _mod_kernel,
        grid_spec=pltpu.PrefetchScalarGridSpec(
            num_scalar_prefetch=1,
            grid=grid,
            in_specs=[
                pl.BlockSpec(memory_space=pl.ANY),
                pl.BlockSpec((num_out, code_dim), lambda i, idx_ref: (0, 0)),
                pl.BlockSpec((1, num_out), lambda i, idx_ref: (0, 0)),
            ],
            out_specs=pl.BlockSpec((_ROWS_PER_TILE, num_out), lambda i, idx_ref: (i, 0)),
            scratch_shapes=[
                pltpu.VMEM((128,), jnp.float32),
                pltpu.SemaphoreType.DMA,
            ],
        ),
        out_shape=jax.ShapeDtypeStruct((n, num_out), jnp.float32),
    )(idx, flat, W, b.reshape(1, num_out))
    return out


# speedup vs baseline: 90.1731x; 90.1731x over previous
import jax
import jax.numpy as jnp
from jax.experimental import pallas as pl
from jax.experimental.pallas import tpu as pltpu

_ROWS_PER_TILE = 2048
_LANES = 128


def _mod_kernel(idx_ref, tab_ref, WT_ref, b_ref, out_ref):
    lane = idx_ref[0] % _LANES
    block = tab_ref[...]
    sel = (jax.lax.broadcasted_iota(jnp.int32, block.shape, 1) == lane)
    code = jnp.sum(jnp.where(sel, block, 0.0), axis=1)
    mods = jnp.dot(code, WT_ref[...], preferred_element_type=jnp.float32)
    mods = mods + b_ref[0, :]
    out_ref[...] = jnp.broadcast_to(mods[None, :], out_ref.shape)


def kernel(coords, patient_idx, emb_table, W, b):
    n = coords.shape[0]
    num_out, code_dim = W.shape
    idx = jnp.asarray(patient_idx, jnp.int32).reshape((1,))
    tabT = emb_table.T
    WT = W.T
    grid = (n // _ROWS_PER_TILE,)
    out = pl.pallas_call(
        _mod_kernel,
        grid_spec=pltpu.PrefetchScalarGridSpec(
            num_scalar_prefetch=1,
            grid=grid,
            in_specs=[
                pl.BlockSpec((code_dim, _LANES), lambda i, idx_ref: (0, idx_ref[0] // _LANES)),
                pl.BlockSpec((code_dim, num_out), lambda i, idx_ref: (0, 0)),
                pl.BlockSpec((1, num_out), lambda i, idx_ref: (0, 0)),
            ],
            out_specs=pl.BlockSpec((_ROWS_PER_TILE, num_out), lambda i, idx_ref: (i, 0)),
        ),
        out_shape=jax.ShapeDtypeStruct((n, num_out), jnp.float32),
    )(idx, tabT, WT, b.reshape(1, num_out))
    return out


# speedup vs baseline: 102.6168x; 1.1380x over previous
import jax
import jax.numpy as jnp
from jax.experimental import pallas as pl
from jax.experimental.pallas import tpu as pltpu

_ROWS_PER_TILE = 2048
_LANES = 128


def _mod_kernel(idx_ref, tab_ref, WT_ref, b_ref, out_ref):
    i = pl.program_id(0)

    @pl.when(i < 2)
    def _fill():
        lane = idx_ref[0] % _LANES
        block = tab_ref[...]
        sel = (jax.lax.broadcasted_iota(jnp.int32, block.shape, 1) == lane)
        code = jnp.sum(jnp.where(sel, block, 0.0), axis=1)
        mods = jnp.dot(code, WT_ref[...], preferred_element_type=jnp.float32)
        mods = mods + b_ref[0, :]
        out_ref[...] = jnp.broadcast_to(mods[None, :], out_ref.shape)


def kernel(coords, patient_idx, emb_table, W, b):
    n = coords.shape[0]
    num_out, code_dim = W.shape
    idx = jnp.asarray(patient_idx, jnp.int32).reshape((1,))
    tabT = emb_table.T
    WT = W.T
    grid = (n // _ROWS_PER_TILE,)
    out = pl.pallas_call(
        _mod_kernel,
        grid_spec=pltpu.PrefetchScalarGridSpec(
            num_scalar_prefetch=1,
            grid=grid,
            in_specs=[
                pl.BlockSpec((code_dim, _LANES), lambda i, idx_ref: (0, idx_ref[0] // _LANES)),
                pl.BlockSpec((code_dim, num_out), lambda i, idx_ref: (0, 0)),
                pl.BlockSpec((1, num_out), lambda i, idx_ref: (0, 0)),
            ],
            out_specs=pl.BlockSpec((_ROWS_PER_TILE, num_out), lambda i, idx_ref: (i, 0)),
        ),
        out_shape=jax.ShapeDtypeStruct((n, num_out), jnp.float32),
    )(idx, tabT, WT, b.reshape(1, num_out))
    return out
